# baseline (speedup 1.0000x reference)
import functools

import jax
import jax.numpy as jnp
from jax.experimental import pallas as pl

NTYPE = 3
NHID = 64
NCLASS = 16
GAMMA = 0.1
BR = 200


def _leaky(x):
    return jnp.where(x >= 0, x, 0.2 * x)


def _rowdot(m, v_row):
    return jax.lax.dot_general(m, v_row, (((1,), (1,)), ((), ())),
                               preferred_element_type=jnp.float32)


def _conv_gc1_kernel(x_ref, w1_ref, b1_ref, w2_ref, b2_ref, gw_ref, o_ref):
    x = x_ref[...]
    r = x.shape[0]
    z = jnp.zeros((r, 1), jnp.float32)
    xp = jnp.concatenate([z, x, z], axis=1)
    w1 = w1_ref[...]
    b1 = b1_ref[...]
    chans1 = []
    for c in range(2):
        y = (w1[c, 0, 0] * xp[:, 0:128] + w1[c, 0, 1] * xp[:, 1:129]
             + w1[c, 0, 2] * xp[:, 2:130] + b1[0, c])
        y = jnp.maximum(y, 0.0)
        y = y.reshape(r, 64, 2).max(axis=-1)
        chans1.append(y)
    w2 = w2_ref[...]
    b2 = b2_ref[...]
    zp = jnp.zeros((r, 1), jnp.float32)
    cp = [jnp.concatenate([zp, ch, zp], axis=1) for ch in chans1]
    outs = []
    for c in range(4):
        acc = b2[0, c]
        for ic in range(2):
            acc = acc + (w2[c, ic, 0] * cp[ic][:, 0:64]
                         + w2[c, ic, 1] * cp[ic][:, 1:65]
                         + w2[c, ic, 2] * cp[ic][:, 2:66])
        acc = jnp.maximum(acc, 0.0)
        acc = acc.reshape(r, 32, 2).max(axis=-1)
        outs.append(acc)
    feat = jnp.concatenate(outs, axis=1)
    o_ref[...] = jnp.dot(feat, gw_ref[...], preferred_element_type=jnp.float32)


def _gc1_kernel(x_ref, gw_ref, o_ref):
    o_ref[...] = jnp.dot(x_ref[...], gw_ref[...],
                         preferred_element_type=jnp.float32)


def _stage2_kernel(h_self_ref,
                   adj0_ref, adj1_ref, adj2_ref,
                   h0_ref, h1_ref, h2_ref,
                   a1_ref, a2_ref,
                   w_ref, b_ref, aa_ref,
                   o_ref, *, t1):
    h_self_blk = h_self_ref[...]
    adjs = (adj0_ref[...], adj1_ref[...], adj2_ref[...])
    hs = (h0_ref[...], h1_ref[...], h2_ref[...])
    outs = []
    for t2 in range(NTYPE):
        adj = adjs[t2]
        h2 = hs[t2]
        a1 = a1_ref[:, t2 * NHID:(t2 + 1) * NHID]
        a2 = a2_ref[:, t2 * NHID:(t2 + 1) * NHID]
        r = _rowdot(h_self_blk, a1)
        cT = jax.lax.dot_general(a2, h2, (((1,), (1,)), ((), ())),
                                 preferred_element_type=jnp.float32)
        e = _leaky(r + cT)
        m = _leaky(r + jnp.max(cT, axis=1, keepdims=True))
        p = jnp.where(adj > 0, jnp.exp(e - m), 0.0)
        denom = jnp.sum(p, axis=1, keepdims=True)
        mix = p * (GAMMA / denom) + (1.0 - GAMMA) * adj
        outs.append(jnp.dot(mix, h2, preferred_element_type=jnp.float32))
    w = w_ref[...]
    b = b_ref[...]
    a_top = aa_ref[:, :50]
    a_bot = aa_ref[:, 50:]
    hh = [jnp.tanh(jnp.dot(o, w, preferred_element_type=jnp.float32) + b)
          for o in outs]
    e_self = _rowdot(hh[t1], a_top)
    es = [_leaky(e_self + _rowdot(hh[t], a_bot)) for t in range(NTYPE)]
    mx = jnp.maximum(jnp.maximum(es[0], es[1]), es[2])
    ws = [jnp.exp(e - mx) for e in es]
    den = ws[0] + ws[1] + ws[2]
    xt = (ws[0] * outs[0] + ws[1] * outs[1] + ws[2] * outs[2]) / den
    o_ref[...] = jnp.maximum(xt, 0.0)


def _gc2_kernel(x_ref, w_ref, o_ref):
    o_ref[...] = jnp.dot(x_ref[...], w_ref[...],
                         preferred_element_type=jnp.float32)


def _stage3_kernel(adj0_ref, adj1_ref, adj2_ref,
                   y0_ref, y1_ref, y2_ref,
                   gb_ref, w_ref, b_ref, aa_ref,
                   o_ref, *, t1):
    adjs = (adj0_ref[...], adj1_ref[...], adj2_ref[...])
    ys = (y0_ref[...], y1_ref[...], y2_ref[...])
    gb = gb_ref[...]
    outs = [jnp.dot(adjs[t], ys[t], preferred_element_type=jnp.float32) + gb
            for t in range(NTYPE)]
    w = w_ref[...]
    b = b_ref[...]
    a_top = aa_ref[:, :50]
    a_bot = aa_ref[:, 50:]
    hh = [jnp.tanh(jnp.dot(o, w, preferred_element_type=jnp.float32) + b)
          for o in outs]
    e_self = _rowdot(hh[t1], a_top)
    es = [_leaky(e_self + _rowdot(hh[t], a_bot)) for t in range(NTYPE)]
    mx = jnp.maximum(jnp.maximum(es[0], es[1]), es[2])
    ws = [jnp.exp(e - mx) for e in es]
    den = ws[0] + ws[1] + ws[2]
    xt = (ws[0] * outs[0] + ws[1] * outs[1] + ws[2] * outs[2]) / den
    m = jnp.max(xt, axis=1, keepdims=True)
    sh = xt - m
    lse = jnp.log(jnp.sum(jnp.exp(sh), axis=1, keepdims=True))
    o_ref[...] = sh - lse


def _full(shape):
    return pl.BlockSpec(shape, lambda i: (0,) * len(shape))


def _rows(shape):
    return pl.BlockSpec(shape, lambda i: (i,) + (0,) * (len(shape) - 1))


@jax.jit
def kernel(x_0, x_1, x_2, adj_00, adj_01, adj_02, adj_10, adj_11, adj_12,
           adj_20, adj_21, adj_22, conv1_w, conv1_b, conv2_w, conv2_b,
           gc1_W_0, gc1_W_1, gc1_W_2, att_a1_0, att_a1_1, att_a1_2,
           att_a2_0, att_a2_1, att_a2_2, at1_W_0, at1_W_1, at1_W_2,
           at1_b_0, at1_b_1, at1_b_2, at1_a_0, at1_a_1, at1_a_2,
           at2_W_0, at2_W_1, at2_W_2, at2_b_0, at2_b_1, at2_b_2,
           at2_a_0, at2_a_1, at2_a_2, gc2_W, gc2_b):
    xs = (x_0, x_1, x_2)
    adj = ((adj_00, adj_01, adj_02), (adj_10, adj_11, adj_12),
           (adj_20, adj_21, adj_22))
    Ns = tuple(x.shape[0] for x in xs)
    gc1 = (gc1_W_0, gc1_W_1, gc1_W_2)
    c1b = conv1_b.reshape(1, 2)
    c2b = conv2_b.reshape(1, 4)

    h = []
    for t in range(NTYPE):
        n = Ns[t]
        if t != 1:
            h_t = pl.pallas_call(
                _conv_gc1_kernel,
                grid=(n // BR,),
                in_specs=[_rows((BR, 128)), _full((2, 1, 3)), _full((1, 2)),
                          _full((4, 2, 3)), _full((1, 4)), _full((128, NHID))],
                out_specs=_rows((BR, NHID)),
                out_shape=jax.ShapeDtypeStruct((n, NHID), jnp.float32),
            )(xs[t], conv1_w, c1b, conv2_w, c2b, gc1[t])
        else:
            h_t = pl.pallas_call(
                _gc1_kernel,
                grid=(n // BR,),
                in_specs=[_rows((BR, 128)), _full((128, NHID))],
                out_specs=_rows((BR, NHID)),
                out_shape=jax.ShapeDtypeStruct((n, NHID), jnp.float32),
            )(xs[t], gc1[t])
        h.append(h_t)

    a1_cat = jnp.concatenate(
        [a.reshape(1, NHID) for a in (att_a1_0, att_a1_1, att_a1_2)], axis=1)
    a2_cat = jnp.concatenate(
        [a.reshape(1, NHID) for a in (att_a2_0, att_a2_1, att_a2_2)], axis=1)
    at1_W = (at1_W_0, at1_W_1, at1_W_2)
    at1_b = (at1_b_0, at1_b_1, at1_b_2)
    at1_a = (at1_a_0, at1_a_1, at1_a_2)

    x1 = []
    for t1 in range(NTYPE):
        n = Ns[t1]
        x1_t = pl.pallas_call(
            functools.partial(_stage2_kernel, t1=t1),
            grid=(n // BR,),
            in_specs=[_rows((BR, NHID)),
                      _rows((BR, Ns[0])), _rows((BR, Ns[1])),
                      _rows((BR, Ns[2])),
                      _full((Ns[0], NHID)), _full((Ns[1], NHID)),
                      _full((Ns[2], NHID)),
                      _full((1, 3 * NHID)), _full((1, 3 * NHID)),
                      _full((NHID, 50)), _full((1, 50)), _full((1, 100))],
            out_specs=_rows((BR, NHID)),
            out_shape=jax.ShapeDtypeStruct((n, NHID), jnp.float32),
        )(h[t1], adj[t1][0], adj[t1][1], adj[t1][2], h[0], h[1], h[2],
          a1_cat, a2_cat, at1_W[t1], at1_b[t1].reshape(1, 50),
          at1_a[t1].reshape(1, 100))
        x1.append(x1_t)

    ys = []
    for t in range(NTYPE):
        n = Ns[t]
        y_t = pl.pallas_call(
            _gc2_kernel,
            grid=(1,),
            in_specs=[_full((n, NHID)), _full((NHID, NCLASS))],
            out_specs=_full((n, NCLASS)),
            out_shape=jax.ShapeDtypeStruct((n, NCLASS), jnp.float32),
        )(x1[t], gc2_W)
        ys.append(y_t)

    at2_W = (at2_W_0, at2_W_1, at2_W_2)
    at2_b = (at2_b_0, at2_b_1, at2_b_2)
    at2_a = (at2_a_0, at2_a_1, at2_a_2)
    gb = gc2_b.reshape(1, NCLASS)
    x2 = []
    for t1 in range(NTYPE):
        n = Ns[t1]
        x2_t = pl.pallas_call(
            functools.partial(_stage3_kernel, t1=t1),
            grid=(n // BR,),
            in_specs=[_rows((BR, Ns[0])), _rows((BR, Ns[1])),
                      _rows((BR, Ns[2])),
                      _full((Ns[0], NCLASS)), _full((Ns[1], NCLASS)),
                      _full((Ns[2], NCLASS)),
                      _full((1, NCLASS)), _full((NCLASS, 50)),
                      _full((1, 50)), _full((1, 100))],
            out_specs=_rows((BR, NCLASS)),
            out_shape=jax.ShapeDtypeStruct((n, NCLASS), jnp.float32),
        )(adj[t1][0], adj[t1][1], adj[t1][2], ys[0], ys[1], ys[2],
          gb, at2_W[t1], at2_b[t1].reshape(1, 50), at2_a[t1].reshape(1, 100))
        x2.append(x2_t)
    return tuple(x2)
